# baseline (speedup 1.0000x reference)
import functools

import jax
from jax import lax
import jax.numpy as jnp
from jax.experimental import pallas as pl
from jax.experimental.pallas import tpu as pltpu
from jax.experimental.pallas import tpu_sc as plsc

_FEATURE_DIMS = [64, 64, 128, 256]
_K = 16
_B, _N = 8, 1024
_BNK = _B * _N * _K

_NW = 32
_CH = 128


def _sc_gather(hflat, gidx, dpad):
    e_total = _BNK
    per_w = e_total // _NW
    nch = per_w // _CH
    mesh = plsc.VectorSubcoreMesh(core_axis_name="c", subcore_axis_name="s")

    @functools.partial(
        pl.kernel, mesh=mesh,
        out_type=jax.ShapeDtypeStruct((e_total, dpad), jnp.float32),
        scratch_types=[
            pltpu.VMEM((nch, _CH), jnp.int32),
            pltpu.VMEM((_CH, dpad), jnp.float32),
            pltpu.VMEM((_CH, dpad), jnp.float32),
            pltpu.SemaphoreType.DMA,
            pltpu.SemaphoreType.DMA,
        ],
    )
    def k(h_hbm, idx_hbm, out_hbm, idx_v, buf0, buf1, gsem, wsem):
        wid = lax.axis_index("s") * 2 + lax.axis_index("c")
        base = wid * per_w
        pltpu.sync_copy(idx_hbm.at[wid], idx_v)

        def wait_gather(buf):
            pltpu.make_async_copy(h_hbm.at[pl.ds(0, _CH)], buf, gsem).wait()

        pltpu.async_copy(h_hbm.at[idx_v.at[0]], buf0, gsem)

        def body(jj, _):
            j0 = jj * 2
            pltpu.async_copy(h_hbm.at[idx_v.at[j0 + 1]], buf1, gsem)
            wait_gather(buf0)
            pltpu.async_copy(
                buf0, out_hbm.at[pl.ds(base + j0 * _CH, _CH)], wsem).wait()

            @pl.when(jj + 1 < nch // 2)
            def _():
                pltpu.async_copy(h_hbm.at[idx_v.at[j0 + 2]], buf0, gsem)
            wait_gather(buf1)
            pltpu.async_copy(
                buf1, out_hbm.at[pl.ds(base + (j0 + 1) * _CH, _CH)],
                wsem).wait()
            return 0

        lax.fori_loop(0, nch // 2, body, 0)

    return k(hflat, gidx)


_TBLK = 128
_TCHK = 128


def _topk_kernel(h_ref, hc_ref, idx_ref, d_ref, mn_ref):
    b = pl.program_id(0)
    h = h_ref[0]
    hc = hc_ref[0]
    din = h.shape[1]
    hh = jax.lax.dot_general(h, hc, (((1,), (1,)), ((), ())),
                             preferred_element_type=jnp.float32)
    h2 = h * h
    ones_col = jnp.ones((din, 1), jnp.float32)
    sq_col = jnp.dot(h2, ones_col,
                     preferred_element_type=jnp.float32)
    ones_row = jnp.ones((1, din), jnp.float32)
    sqc_row = jax.lax.dot_general(ones_row, hc * hc,
                                  (((1,), (1,)), ((), ())),
                                  preferred_element_type=jnp.float32)
    d_ref[...] = (sqc_row - 2.0 * hh) + sq_col
    base = b * _N
    nch = _N // _TCHK
    riota = jax.lax.broadcasted_iota(jnp.int32, (_TCHK, _TBLK), 0)

    mn = jnp.full((1, _TBLK), jnp.inf, jnp.float32)
    for g in range(nch):
        ch = d_ref[pl.ds(g * _TCHK, _TCHK), :]
        mn = jnp.minimum(mn, jnp.min(ch, axis=0, keepdims=True))

    for it in range(_K):
        big = jnp.float32(3.0e38 - it * 1.0e31)
        am = jnp.full((1, _TBLK), _N, jnp.int32)
        nmn = jnp.full((1, _TBLK), jnp.inf, jnp.float32)
        for g in range(nch):
            ch = d_ref[pl.ds(g * _TCHK, _TCHK), :]
            mask = ch == mn
            am = jnp.minimum(
                am, jnp.min(jnp.where(mask, riota + g * _TCHK, _N),
                            axis=0, keepdims=True))
            tomb = jnp.where(mask, big, ch)
            d_ref[pl.ds(g * _TCHK, _TCHK), :] = tomb
            nmn = jnp.minimum(nmn, jnp.min(tomb, axis=0, keepdims=True))
        idx_ref[0, it, :] = am[0] + base
        mn_ref[...] = nmn

        @pl.when(jnp.any(nmn == mn))
        def _(mn=mn, am=am, big=big):
            nmn2 = jnp.full((1, _TBLK), jnp.inf, jnp.float32)
            for g in range(nch):
                ch = d_ref[pl.ds(g * _TCHK, _TCHK), :]
                restored = jnp.where(ch == big, mn, ch)
                tomb = jnp.where(riota + g * _TCHK == am, big, restored)
                d_ref[pl.ds(g * _TCHK, _TCHK), :] = tomb
                nmn2 = jnp.minimum(nmn2, jnp.min(tomb, axis=0,
                                                 keepdims=True))
            mn_ref[...] = nmn2

        mn = mn_ref[...]


def _topk(h):
    din = h.shape[-1]
    return pl.pallas_call(
        _topk_kernel,
        grid=(_B, _N // _TBLK),
        in_specs=[
            pl.BlockSpec((1, _N, din), lambda b, c: (b, 0, 0)),
            pl.BlockSpec((1, _TBLK, din), lambda b, c: (b, c, 0)),
        ],
        out_specs=pl.BlockSpec((1, _K, _TBLK), lambda b, c: (b, 0, c)),
        out_shape=jax.ShapeDtypeStruct((_B, _K, _N), jnp.int32),
        scratch_shapes=[pltpu.VMEM((_N, _TBLK), jnp.float32),
                        pltpu.VMEM((1, _TBLK), jnp.float32)],
    )(h, h)


def _edge_kernel(hj_ref, hi_ref, wt_ref, wp_ref, bt_ref, bp_ref,
                 mmax_ref, s_ref):
    blk = hi_ref.shape[1]
    din = hi_ref.shape[2]
    dout = wt_ref.shape[1]
    hi = hi_ref[0]
    hj = hj_ref[0]
    diff = hi[None, :, :] - hj
    t1 = jax.lax.dot_general(diff.reshape(_K * blk, din), wt_ref[...],
                             (((1,), (0,)), ((), ())),
                             preferred_element_type=jnp.float32)
    t2 = jax.lax.dot_general(hi, wp_ref[...], (((1,), (0,)), ((), ())),
                             preferred_element_type=jnp.float32)
    m = ((t1.reshape(_K, blk, dout) + bt_ref[...]) + t2[None, :, :]) \
        + bp_ref[...]
    mmax_ref[0] = jnp.max(m, axis=0)

    step = pl.program_id(0) * pl.num_programs(1) + pl.program_id(1)
    @pl.when(step == 0)
    def _():
        s_ref[...] = jnp.zeros_like(s_ref)
    s1 = jnp.sum(m, axis=(0, 1))
    s2 = jnp.sum(m * m, axis=(0, 1))
    s_ref[...] += jnp.stack([s1, s2], axis=0)


def _edge_mlp(hj, h, wt, wp, bt, bp, blk=256):
    din = h.shape[-1]
    dout = wt.shape[0]
    grid = (_B, _N // blk)
    mmax, s = pl.pallas_call(
        _edge_kernel,
        grid=grid,
        in_specs=[
            pl.BlockSpec((1, _K, blk, din), lambda b, n: (b, 0, n, 0)),
            pl.BlockSpec((1, blk, din), lambda b, n: (b, n, 0)),
            pl.BlockSpec((din, dout), lambda b, n: (0, 0)),
            pl.BlockSpec((din, dout), lambda b, n: (0, 0)),
            pl.BlockSpec((1, dout), lambda b, n: (0, 0)),
            pl.BlockSpec((1, dout), lambda b, n: (0, 0)),
        ],
        out_specs=[
            pl.BlockSpec((1, blk, dout), lambda b, n: (b, n, 0)),
            pl.BlockSpec((2, dout), lambda b, n: (0, 0)),
        ],
        out_shape=[
            jax.ShapeDtypeStruct((_B, _N, dout), jnp.float32),
            jax.ShapeDtypeStruct((2, dout), jnp.float32),
        ],
    )(hj, h, wt.T, wp.T, bt[None, :], bp[None, :])
    return mmax, s


def _bn_kernel(mmax_ref, s_ref, out_ref):
    dout = mmax_ref.shape[2]
    mu = s_ref[0, :] / _BNK
    var = s_ref[1, :] / _BNK - mu * mu
    hn = (mmax_ref[0] - mu[None, :]) / jnp.sqrt(var + 1e-5)[None, :]
    hn = jnp.where(hn >= 0, hn, 0.2 * hn)
    dpad = out_ref.shape[2]
    if dpad > dout:
        hn = jnp.pad(hn, ((0, 0), (0, dpad - dout)))
    out_ref[0] = hn


def _bn_leaky(mmax, s, dpad):
    dout = mmax.shape[-1]
    return pl.pallas_call(
        _bn_kernel,
        grid=(_B,),
        in_specs=[
            pl.BlockSpec((1, _N, dout), lambda b: (b, 0, 0)),
            pl.BlockSpec((2, dout), lambda b: (0, 0)),
        ],
        out_specs=pl.BlockSpec((1, _N, dpad), lambda b: (b, 0, 0)),
        out_shape=jax.ShapeDtypeStruct((_B, _N, dpad), jnp.float32),
    )(mmax, s)


def _proj_pool_kernel(h_ref, w_ref, b_ref, out_ref):
    h = h_ref[0]
    p = jnp.dot(h, w_ref[...], preferred_element_type=jnp.float32) + b_ref[...]
    pmax = jnp.max(p, axis=0, keepdims=True)
    pavg = jnp.mean(p, axis=0, keepdims=True)
    out_ref[0] = jnp.concatenate([pmax, pavg], axis=1)


def _proj_pool(h, pw, pb):
    out = pl.pallas_call(
        _proj_pool_kernel,
        grid=(_B,),
        in_specs=[
            pl.BlockSpec((1, _N, 512), lambda b: (b, 0, 0)),
            pl.BlockSpec((512, 512), lambda b: (0, 0)),
            pl.BlockSpec((1, 512), lambda b: (0, 0)),
        ],
        out_specs=pl.BlockSpec((1, 1, 1024), lambda b: (b, 0, 0)),
        out_shape=jax.ShapeDtypeStruct((_B, 1, 1024), jnp.float32),
    )(h, pw.T, pb[None, :])
    return out.reshape(_B, 1024)


def kernel(x, params):
    h = jnp.pad(x, ((0, 0), (0, 0), (0, 125)))
    hs = []
    for i, dout in enumerate(_FEATURE_DIMS):
        wt = params['theta_w_%d' % i]
        bt = params['theta_b_%d' % i]
        wp = params['phi_w_%d' % i]
        bp = params['phi_b_%d' % i]
        din = h.shape[-1]
        if wt.shape[1] != din:
            pad = din - wt.shape[1]
            wt = jnp.pad(wt, ((0, 0), (0, pad)))
            wp = jnp.pad(wp, ((0, 0), (0, pad)))

        gidx = _topk(h)
        hflat = h.reshape(_B * _N, din)
        nch = _BNK // _NW // _CH
        hj = _sc_gather(hflat, gidx.reshape(_NW, nch, _CH), din)
        hj = hj.reshape(_B, _K, _N, din)
        mmax, s = _edge_mlp(hj, h, wt, wp, bt, bp)
        h = _bn_leaky(mmax, s, max(dout, 128))
        hs.append(h[:, :, :dout])

    hcat = jnp.concatenate(hs, axis=2)
    return _proj_pool(hcat, params['proj_w'], params['proj_b'])


# speedup vs baseline: 1.3840x; 1.3840x over previous
import functools

import jax
from jax import lax
import jax.numpy as jnp
from jax.experimental import pallas as pl
from jax.experimental.pallas import tpu as pltpu
from jax.experimental.pallas import tpu_sc as plsc

_FEATURE_DIMS = [64, 64, 128, 256]
_K = 16
_B, _N = 8, 1024
_BNK = _B * _N * _K

_NW = 32
_CH = 128


def _sc_gather(hflat, gidx, dpad):
    e_total = _BNK
    per_w = e_total // _NW
    nch = per_w // _CH
    mesh = plsc.VectorSubcoreMesh(core_axis_name="c", subcore_axis_name="s")

    @functools.partial(
        pl.kernel, mesh=mesh,
        out_type=jax.ShapeDtypeStruct((e_total, dpad), jnp.float32),
        scratch_types=[
            pltpu.VMEM((nch, _CH), jnp.int32),
            pltpu.VMEM((_CH, dpad), jnp.float32),
            pltpu.VMEM((_CH, dpad), jnp.float32),
            pltpu.SemaphoreType.DMA,
            pltpu.SemaphoreType.DMA,
        ],
    )
    def k(h_hbm, idx_hbm, out_hbm, idx_v, buf0, buf1, gsem, wsem):
        wid = lax.axis_index("s") * 2 + lax.axis_index("c")
        base = wid * per_w
        pltpu.sync_copy(idx_hbm.at[wid], idx_v)

        def wait_gather(buf):
            pltpu.make_async_copy(h_hbm.at[pl.ds(0, _CH)], buf, gsem).wait()

        pltpu.async_copy(h_hbm.at[idx_v.at[0]], buf0, gsem)

        def body(jj, _):
            j0 = jj * 2
            pltpu.async_copy(h_hbm.at[idx_v.at[j0 + 1]], buf1, gsem)
            wait_gather(buf0)
            pltpu.async_copy(
                buf0, out_hbm.at[pl.ds(base + j0 * _CH, _CH)], wsem).wait()

            @pl.when(jj + 1 < nch // 2)
            def _():
                pltpu.async_copy(h_hbm.at[idx_v.at[j0 + 2]], buf0, gsem)
            wait_gather(buf1)
            pltpu.async_copy(
                buf1, out_hbm.at[pl.ds(base + (j0 + 1) * _CH, _CH)],
                wsem).wait()
            return 0

        lax.fori_loop(0, nch // 2, body, 0)

    return k(hflat, gidx)


_TBLK = 128
_TCHK = 128


def _topk_kernel(h_ref, hc_ref, idx_ref, d_ref, mn_ref):
    b = pl.program_id(0)
    h = h_ref[0]
    hc = hc_ref[0]
    din = h.shape[1]
    hh = jax.lax.dot_general(h, hc, (((1,), (1,)), ((), ())),
                             preferred_element_type=jnp.float32)
    h2 = h * h
    ones_col = jnp.ones((din, 1), jnp.float32)
    sq_col = jnp.dot(h2, ones_col,
                     preferred_element_type=jnp.float32)
    ones_row = jnp.ones((1, din), jnp.float32)
    sqc_row = jax.lax.dot_general(ones_row, hc * hc,
                                  (((1,), (1,)), ((), ())),
                                  preferred_element_type=jnp.float32)
    d_ref[...] = (sqc_row - 2.0 * hh) + sq_col
    base = b * _N
    nch = _N // _TCHK
    riota = jax.lax.broadcasted_iota(jnp.int32, (_TCHK, _TBLK), 0)

    mn = jnp.full((1, _TBLK), jnp.inf, jnp.float32)
    for g in range(nch):
        ch = d_ref[pl.ds(g * _TCHK, _TCHK), :]
        mn = jnp.minimum(mn, jnp.min(ch, axis=0, keepdims=True))

    for it in range(_K):
        am = jnp.full((1, _TBLK), _N, jnp.int32)
        for g in range(nch):
            ch = d_ref[pl.ds(g * _TCHK, _TCHK), :]
            cand = jnp.where(ch == mn, riota + g * _TCHK, _N)
            am = jnp.minimum(am, jnp.min(cand, axis=0, keepdims=True))
        idx_ref[0, it, :] = am[0] + base
        if it + 1 < _K:
            mn = jnp.full((1, _TBLK), jnp.inf, jnp.float32)
            for g in range(nch):
                ch = d_ref[pl.ds(g * _TCHK, _TCHK), :]
                masked = jnp.where(riota + g * _TCHK == am, jnp.inf, ch)
                d_ref[pl.ds(g * _TCHK, _TCHK), :] = masked
                mn = jnp.minimum(mn, jnp.min(masked, axis=0, keepdims=True))


def _topk(h):
    din = h.shape[-1]
    return pl.pallas_call(
        _topk_kernel,
        grid=(_B, _N // _TBLK),
        in_specs=[
            pl.BlockSpec((1, _N, din), lambda b, c: (b, 0, 0)),
            pl.BlockSpec((1, _TBLK, din), lambda b, c: (b, c, 0)),
        ],
        out_specs=pl.BlockSpec((1, _K, _TBLK), lambda b, c: (b, 0, c)),
        out_shape=jax.ShapeDtypeStruct((_B, _K, _N), jnp.int32),
        scratch_shapes=[pltpu.VMEM((_N, _TBLK), jnp.float32),
                        pltpu.VMEM((1, _TBLK), jnp.float32)],
    )(h, h)


def _edge_kernel(hj_ref, hi_ref, wt_ref, wp_ref, bt_ref, bp_ref,
                 mmax_ref, s_ref):
    blk = hi_ref.shape[1]
    din = hi_ref.shape[2]
    dout = wt_ref.shape[1]
    hi = hi_ref[0]
    hj = hj_ref[0]
    diff = hi[None, :, :] - hj
    t1 = jax.lax.dot_general(diff.reshape(_K * blk, din), wt_ref[...],
                             (((1,), (0,)), ((), ())),
                             preferred_element_type=jnp.float32)
    t2 = jax.lax.dot_general(hi, wp_ref[...], (((1,), (0,)), ((), ())),
                             preferred_element_type=jnp.float32)
    m = ((t1.reshape(_K, blk, dout) + bt_ref[...]) + t2[None, :, :]) \
        + bp_ref[...]
    mmax_ref[0] = jnp.max(m, axis=0)

    step = pl.program_id(0) * pl.num_programs(1) + pl.program_id(1)
    @pl.when(step == 0)
    def _():
        s_ref[...] = jnp.zeros_like(s_ref)
    s1 = jnp.sum(m, axis=(0, 1))
    s2 = jnp.sum(m * m, axis=(0, 1))
    s_ref[...] += jnp.stack([s1, s2], axis=0)


def _edge_mlp(hj, h, wt, wp, bt, bp, blk=256):
    din = h.shape[-1]
    dout = wt.shape[0]
    grid = (_B, _N // blk)
    mmax, s = pl.pallas_call(
        _edge_kernel,
        grid=grid,
        in_specs=[
            pl.BlockSpec((1, _K, blk, din), lambda b, n: (b, 0, n, 0)),
            pl.BlockSpec((1, blk, din), lambda b, n: (b, n, 0)),
            pl.BlockSpec((din, dout), lambda b, n: (0, 0)),
            pl.BlockSpec((din, dout), lambda b, n: (0, 0)),
            pl.BlockSpec((1, dout), lambda b, n: (0, 0)),
            pl.BlockSpec((1, dout), lambda b, n: (0, 0)),
        ],
        out_specs=[
            pl.BlockSpec((1, blk, dout), lambda b, n: (b, n, 0)),
            pl.BlockSpec((2, dout), lambda b, n: (0, 0)),
        ],
        out_shape=[
            jax.ShapeDtypeStruct((_B, _N, dout), jnp.float32),
            jax.ShapeDtypeStruct((2, dout), jnp.float32),
        ],
    )(hj, h, wt.T, wp.T, bt[None, :], bp[None, :])
    return mmax, s


def _bn_kernel(mmax_ref, s_ref, out_ref):
    dout = mmax_ref.shape[2]
    mu = s_ref[0, :] / _BNK
    var = s_ref[1, :] / _BNK - mu * mu
    hn = (mmax_ref[0] - mu[None, :]) / jnp.sqrt(var + 1e-5)[None, :]
    hn = jnp.where(hn >= 0, hn, 0.2 * hn)
    dpad = out_ref.shape[2]
    if dpad > dout:
        hn = jnp.pad(hn, ((0, 0), (0, dpad - dout)))
    out_ref[0] = hn


def _bn_leaky(mmax, s, dpad):
    dout = mmax.shape[-1]
    return pl.pallas_call(
        _bn_kernel,
        grid=(_B,),
        in_specs=[
            pl.BlockSpec((1, _N, dout), lambda b: (b, 0, 0)),
            pl.BlockSpec((2, dout), lambda b: (0, 0)),
        ],
        out_specs=pl.BlockSpec((1, _N, dpad), lambda b: (b, 0, 0)),
        out_shape=jax.ShapeDtypeStruct((_B, _N, dpad), jnp.float32),
    )(mmax, s)


def _proj_pool_kernel(h_ref, w_ref, b_ref, out_ref):
    h = h_ref[0]
    p = jnp.dot(h, w_ref[...], preferred_element_type=jnp.float32) + b_ref[...]
    pmax = jnp.max(p, axis=0, keepdims=True)
    pavg = jnp.mean(p, axis=0, keepdims=True)
    out_ref[0] = jnp.concatenate([pmax, pavg], axis=1)


def _proj_pool(h, pw, pb):
    out = pl.pallas_call(
        _proj_pool_kernel,
        grid=(_B,),
        in_specs=[
            pl.BlockSpec((1, _N, 512), lambda b: (b, 0, 0)),
            pl.BlockSpec((512, 512), lambda b: (0, 0)),
            pl.BlockSpec((1, 512), lambda b: (0, 0)),
        ],
        out_specs=pl.BlockSpec((1, 1, 1024), lambda b: (b, 0, 0)),
        out_shape=jax.ShapeDtypeStruct((_B, 1, 1024), jnp.float32),
    )(h, pw.T, pb[None, :])
    return out.reshape(_B, 1024)


def kernel(x, params):
    h = jnp.pad(x, ((0, 0), (0, 0), (0, 125)))
    hs = []
    for i, dout in enumerate(_FEATURE_DIMS):
        wt = params['theta_w_%d' % i]
        bt = params['theta_b_%d' % i]
        wp = params['phi_w_%d' % i]
        bp = params['phi_b_%d' % i]
        din = h.shape[-1]
        if wt.shape[1] != din:
            pad = din - wt.shape[1]
            wt = jnp.pad(wt, ((0, 0), (0, pad)))
            wp = jnp.pad(wp, ((0, 0), (0, pad)))

        gidx = _topk(h)
        hflat = h.reshape(_B * _N, din)
        nch = _BNK // _NW // _CH
        hj = _sc_gather(hflat, gidx.reshape(_NW, nch, _CH), din)
        hj = hj.reshape(_B, _K, _N, din)
        mmax, s = _edge_mlp(hj, h, wt, wp, bt, bp)
        h = _bn_leaky(mmax, s, max(dout, 128))
        hs.append(h[:, :, :dout])

    hcat = jnp.concatenate(hs, axis=2)
    return _proj_pool(hcat, params['proj_w'], params['proj_b'])


# speedup vs baseline: 1.3901x; 1.0044x over previous
import functools

import jax
from jax import lax
import jax.numpy as jnp
from jax.experimental import pallas as pl
from jax.experimental.pallas import tpu as pltpu
from jax.experimental.pallas import tpu_sc as plsc

_FEATURE_DIMS = [64, 64, 128, 256]
_K = 16
_B, _N = 8, 1024
_BNK = _B * _N * _K

_NW = 32
_CH = 128


def _sc_gather(hflat, gidx, dpad):
    e_total = _BNK
    per_w = e_total // _NW
    nch = per_w // _CH
    mesh = plsc.VectorSubcoreMesh(core_axis_name="c", subcore_axis_name="s")

    @functools.partial(
        pl.kernel, mesh=mesh,
        out_type=jax.ShapeDtypeStruct((e_total, dpad), jnp.float32),
        scratch_types=[
            pltpu.VMEM((nch, _CH), jnp.int32),
            pltpu.VMEM((_CH, dpad), jnp.float32),
            pltpu.VMEM((_CH, dpad), jnp.float32),
            pltpu.SemaphoreType.DMA,
            pltpu.SemaphoreType.DMA,
        ],
    )
    def k(h_hbm, idx_hbm, out_hbm, idx_v, buf0, buf1, gsem, wsem):
        wid = lax.axis_index("s") * 2 + lax.axis_index("c")
        base = wid * per_w
        pltpu.sync_copy(idx_hbm.at[wid], idx_v)

        def wait_gather(buf):
            pltpu.make_async_copy(h_hbm.at[pl.ds(0, _CH)], buf, gsem).wait()

        pltpu.async_copy(h_hbm.at[idx_v.at[0]], buf0, gsem)

        def body(jj, _):
            j0 = jj * 2
            pltpu.async_copy(h_hbm.at[idx_v.at[j0 + 1]], buf1, gsem)
            wait_gather(buf0)
            pltpu.async_copy(
                buf0, out_hbm.at[pl.ds(base + j0 * _CH, _CH)], wsem).wait()

            @pl.when(jj + 1 < nch // 2)
            def _():
                pltpu.async_copy(h_hbm.at[idx_v.at[j0 + 2]], buf0, gsem)
            wait_gather(buf1)
            pltpu.async_copy(
                buf1, out_hbm.at[pl.ds(base + (j0 + 1) * _CH, _CH)],
                wsem).wait()
            return 0

        lax.fori_loop(0, nch // 2, body, 0)

    return k(hflat, gidx)


_TBLK = 128
_TCHK = 128


def _topk_kernel(h_ref, hc_ref, sqc_ref, sqcol_ref, idx_ref, d_ref, mn_ref):
    b = pl.program_id(0)
    h = h_ref[0]
    hc = hc_ref[0]
    hh = jax.lax.dot_general(h, hc, (((1,), (1,)), ((), ())),
                             preferred_element_type=jnp.float32)
    sqc_row = sqc_ref[0]
    sq_col = sqcol_ref[0]
    d_ref[...] = (sqc_row - 2.0 * hh) + sq_col
    base = b * _N
    nch = _N // _TCHK
    riota = jax.lax.broadcasted_iota(jnp.int32, (_TCHK, _TBLK), 0)

    mn = jnp.full((1, _TBLK), jnp.inf, jnp.float32)
    for g in range(nch):
        ch = d_ref[pl.ds(g * _TCHK, _TCHK), :]
        mn = jnp.minimum(mn, jnp.min(ch, axis=0, keepdims=True))

    for it in range(_K):
        am = jnp.full((1, _TBLK), _N, jnp.int32)
        for g in range(nch):
            ch = d_ref[pl.ds(g * _TCHK, _TCHK), :]
            cand = jnp.where(ch == mn, riota + g * _TCHK, _N)
            am = jnp.minimum(am, jnp.min(cand, axis=0, keepdims=True))
        idx_ref[0, it, :] = am[0] + base
        if it + 1 < _K:
            mn = jnp.full((1, _TBLK), jnp.inf, jnp.float32)
            for g in range(nch):
                ch = d_ref[pl.ds(g * _TCHK, _TCHK), :]
                masked = jnp.where(riota + g * _TCHK == am, jnp.inf, ch)
                d_ref[pl.ds(g * _TCHK, _TCHK), :] = masked
                mn = jnp.minimum(mn, jnp.min(masked, axis=0, keepdims=True))


def _topk(h):
    din = h.shape[-1]
    sq = jnp.sum(h * h, axis=-1)
    return pl.pallas_call(
        _topk_kernel,
        grid=(_B, _N // _TBLK),
        in_specs=[
            pl.BlockSpec((1, _N, din), lambda b, c: (b, 0, 0)),
            pl.BlockSpec((1, _TBLK, din), lambda b, c: (b, c, 0)),
            pl.BlockSpec((1, 1, _TBLK), lambda b, c: (b, 0, c)),
            pl.BlockSpec((1, _N, 1), lambda b, c: (b, 0, 0)),
        ],
        out_specs=pl.BlockSpec((1, _K, _TBLK), lambda b, c: (b, 0, c)),
        out_shape=jax.ShapeDtypeStruct((_B, _K, _N), jnp.int32),
        scratch_shapes=[pltpu.VMEM((_N, _TBLK), jnp.float32),
                        pltpu.VMEM((1, _TBLK), jnp.float32)],
    )(h, h, sq[:, None, :], sq[:, :, None])


def _edge_kernel(hj_ref, hi_ref, wt_ref, wp_ref, bt_ref, bp_ref,
                 mmax_ref, s_ref):
    blk = hi_ref.shape[1]
    din = hi_ref.shape[2]
    dout = wt_ref.shape[1]
    hi = hi_ref[0]
    hj = hj_ref[0]
    diff = hi[None, :, :] - hj
    t1 = jax.lax.dot_general(diff.reshape(_K * blk, din), wt_ref[...],
                             (((1,), (0,)), ((), ())),
                             preferred_element_type=jnp.float32)
    t2 = jax.lax.dot_general(hi, wp_ref[...], (((1,), (0,)), ((), ())),
                             preferred_element_type=jnp.float32)
    m = ((t1.reshape(_K, blk, dout) + bt_ref[...]) + t2[None, :, :]) \
        + bp_ref[...]
    mmax_ref[0] = jnp.max(m, axis=0)

    step = pl.program_id(0) * pl.num_programs(1) + pl.program_id(1)
    @pl.when(step == 0)
    def _():
        s_ref[...] = jnp.zeros_like(s_ref)
    s1 = jnp.sum(m, axis=(0, 1))
    s2 = jnp.sum(m * m, axis=(0, 1))
    s_ref[...] += jnp.stack([s1, s2], axis=0)


def _edge_mlp(hj, h, wt, wp, bt, bp, blk=256):
    din = h.shape[-1]
    dout = wt.shape[0]
    grid = (_B, _N // blk)
    mmax, s = pl.pallas_call(
        _edge_kernel,
        grid=grid,
        in_specs=[
            pl.BlockSpec((1, _K, blk, din), lambda b, n: (b, 0, n, 0)),
            pl.BlockSpec((1, blk, din), lambda b, n: (b, n, 0)),
            pl.BlockSpec((din, dout), lambda b, n: (0, 0)),
            pl.BlockSpec((din, dout), lambda b, n: (0, 0)),
            pl.BlockSpec((1, dout), lambda b, n: (0, 0)),
            pl.BlockSpec((1, dout), lambda b, n: (0, 0)),
        ],
        out_specs=[
            pl.BlockSpec((1, blk, dout), lambda b, n: (b, n, 0)),
            pl.BlockSpec((2, dout), lambda b, n: (0, 0)),
        ],
        out_shape=[
            jax.ShapeDtypeStruct((_B, _N, dout), jnp.float32),
            jax.ShapeDtypeStruct((2, dout), jnp.float32),
        ],
    )(hj, h, wt.T, wp.T, bt[None, :], bp[None, :])
    return mmax, s


def _bn_kernel(mmax_ref, s_ref, out_ref):
    dout = mmax_ref.shape[2]
    mu = s_ref[0, :] / _BNK
    var = s_ref[1, :] / _BNK - mu * mu
    hn = (mmax_ref[0] - mu[None, :]) / jnp.sqrt(var + 1e-5)[None, :]
    hn = jnp.where(hn >= 0, hn, 0.2 * hn)
    dpad = out_ref.shape[2]
    if dpad > dout:
        hn = jnp.pad(hn, ((0, 0), (0, dpad - dout)))
    out_ref[0] = hn


def _bn_leaky(mmax, s, dpad):
    dout = mmax.shape[-1]
    return pl.pallas_call(
        _bn_kernel,
        grid=(_B,),
        in_specs=[
            pl.BlockSpec((1, _N, dout), lambda b: (b, 0, 0)),
            pl.BlockSpec((2, dout), lambda b: (0, 0)),
        ],
        out_specs=pl.BlockSpec((1, _N, dpad), lambda b: (b, 0, 0)),
        out_shape=jax.ShapeDtypeStruct((_B, _N, dpad), jnp.float32),
    )(mmax, s)


def _proj_pool_kernel(h_ref, w_ref, b_ref, out_ref):
    h = h_ref[0]
    p = jnp.dot(h, w_ref[...], preferred_element_type=jnp.float32) + b_ref[...]
    pmax = jnp.max(p, axis=0, keepdims=True)
    pavg = jnp.mean(p, axis=0, keepdims=True)
    out_ref[0] = jnp.concatenate([pmax, pavg], axis=1)


def _proj_pool(h, pw, pb):
    out = pl.pallas_call(
        _proj_pool_kernel,
        grid=(_B,),
        in_specs=[
            pl.BlockSpec((1, _N, 512), lambda b: (b, 0, 0)),
            pl.BlockSpec((512, 512), lambda b: (0, 0)),
            pl.BlockSpec((1, 512), lambda b: (0, 0)),
        ],
        out_specs=pl.BlockSpec((1, 1, 1024), lambda b: (b, 0, 0)),
        out_shape=jax.ShapeDtypeStruct((_B, 1, 1024), jnp.float32),
    )(h, pw.T, pb[None, :])
    return out.reshape(_B, 1024)


def kernel(x, params):
    h = jnp.pad(x, ((0, 0), (0, 0), (0, 125)))
    hs = []
    for i, dout in enumerate(_FEATURE_DIMS):
        wt = params['theta_w_%d' % i]
        bt = params['theta_b_%d' % i]
        wp = params['phi_w_%d' % i]
        bp = params['phi_b_%d' % i]
        din = h.shape[-1]
        if wt.shape[1] != din:
            pad = din - wt.shape[1]
            wt = jnp.pad(wt, ((0, 0), (0, pad)))
            wp = jnp.pad(wp, ((0, 0), (0, pad)))

        gidx = _topk(h)
        hflat = h.reshape(_B * _N, din)
        nch = _BNK // _NW // _CH
        hj = _sc_gather(hflat, gidx.reshape(_NW, nch, _CH), din)
        hj = hj.reshape(_B, _K, _N, din)
        mmax, s = _edge_mlp(hj, h, wt, wp, bt, bp)
        h = _bn_leaky(mmax, s, max(dout, 128))
        hs.append(h[:, :, :dout])

    hcat = jnp.concatenate(hs, axis=2)
    return _proj_pool(hcat, params['proj_w'], params['proj_b'])


# speedup vs baseline: 1.4757x; 1.0616x over previous
import functools

import jax
from jax import lax
import jax.numpy as jnp
from jax.experimental import pallas as pl
from jax.experimental.pallas import tpu as pltpu
from jax.experimental.pallas import tpu_sc as plsc

_FEATURE_DIMS = [64, 64, 128, 256]
_K = 16
_B, _N = 8, 1024
_BNK = _B * _N * _K

_NW = 32
_CH = 128


def _sc_gather(hflat, gidx, dpad):
    nch = gidx.shape[1]
    per_w = nch * _CH
    e_total = _NW * per_w
    mesh = plsc.VectorSubcoreMesh(core_axis_name="c", subcore_axis_name="s")

    @functools.partial(
        pl.kernel, mesh=mesh,
        out_type=jax.ShapeDtypeStruct((e_total, dpad), jnp.float32),
        scratch_types=[
            pltpu.VMEM((nch, _CH), jnp.int32),
            pltpu.VMEM((_CH, dpad), jnp.float32),
            pltpu.VMEM((_CH, dpad), jnp.float32),
            pltpu.SemaphoreType.DMA,
            pltpu.SemaphoreType.DMA,
        ],
    )
    def k(h_hbm, idx_hbm, out_hbm, idx_v, buf0, buf1, gsem, wsem):
        wid = lax.axis_index("s") * 2 + lax.axis_index("c")
        base = wid * per_w
        pltpu.sync_copy(idx_hbm.at[wid], idx_v)

        def wait_gather(buf):
            pltpu.make_async_copy(h_hbm.at[pl.ds(0, _CH)], buf, gsem).wait()

        pltpu.async_copy(h_hbm.at[idx_v.at[0]], buf0, gsem)

        def body(jj, _):
            j0 = jj * 2
            pltpu.async_copy(h_hbm.at[idx_v.at[j0 + 1]], buf1, gsem)
            wait_gather(buf0)
            pltpu.async_copy(
                buf0, out_hbm.at[pl.ds(base + j0 * _CH, _CH)], wsem).wait()

            @pl.when(jj + 1 < nch // 2)
            def _():
                pltpu.async_copy(h_hbm.at[idx_v.at[j0 + 2]], buf0, gsem)
            wait_gather(buf1)
            pltpu.async_copy(
                buf1, out_hbm.at[pl.ds(base + (j0 + 1) * _CH, _CH)],
                wsem).wait()
            return 0

        lax.fori_loop(0, nch // 2, body, 0)

    return k(hflat, gidx)


_TBLK = 128
_TCHK = 128


def _topk_kernel(boff, h_ref, hc_ref, sqc_ref, sqcol_ref, idx_ref, d_ref,
                 mn_ref):
    b = pl.program_id(0) + boff
    h = h_ref[0]
    hc = hc_ref[0]
    hh = jax.lax.dot_general(h, hc, (((1,), (1,)), ((), ())),
                             preferred_element_type=jnp.float32)
    sqc_row = sqc_ref[0]
    sq_col = sqcol_ref[0]
    d_ref[...] = (sqc_row - 2.0 * hh) + sq_col
    base = b * _N
    nch = _N // _TCHK
    riota = jax.lax.broadcasted_iota(jnp.int32, (_TCHK, _TBLK), 0)

    mn = jnp.full((1, _TBLK), jnp.inf, jnp.float32)
    for g in range(nch):
        ch = d_ref[pl.ds(g * _TCHK, _TCHK), :]
        mn = jnp.minimum(mn, jnp.min(ch, axis=0, keepdims=True))

    for it in range(_K):
        am = jnp.full((1, _TBLK), _N, jnp.int32)
        for g in range(nch):
            ch = d_ref[pl.ds(g * _TCHK, _TCHK), :]
            cand = jnp.where(ch == mn, riota + g * _TCHK, _N)
            am = jnp.minimum(am, jnp.min(cand, axis=0, keepdims=True))
        idx_ref[0, it, :] = am[0] + base
        if it + 1 < _K:
            mn = jnp.full((1, _TBLK), jnp.inf, jnp.float32)
            for g in range(nch):
                ch = d_ref[pl.ds(g * _TCHK, _TCHK), :]
                masked = jnp.where(riota + g * _TCHK == am, jnp.inf, ch)
                d_ref[pl.ds(g * _TCHK, _TCHK), :] = masked
                mn = jnp.minimum(mn, jnp.min(masked, axis=0, keepdims=True))


def _topk(h, boff=0):
    bsub = h.shape[0]
    din = h.shape[-1]
    sq = jnp.sum(h * h, axis=-1)
    return pl.pallas_call(
        functools.partial(_topk_kernel, boff),
        grid=(bsub, _N // _TBLK),
        in_specs=[
            pl.BlockSpec((1, _N, din), lambda b, c: (b, 0, 0)),
            pl.BlockSpec((1, _TBLK, din), lambda b, c: (b, c, 0)),
            pl.BlockSpec((1, 1, _TBLK), lambda b, c: (b, 0, c)),
            pl.BlockSpec((1, _N, 1), lambda b, c: (b, 0, 0)),
        ],
        out_specs=pl.BlockSpec((1, _K, _TBLK), lambda b, c: (b, 0, c)),
        out_shape=jax.ShapeDtypeStruct((bsub, _K, _N), jnp.int32),
        scratch_shapes=[pltpu.VMEM((_N, _TBLK), jnp.float32),
                        pltpu.VMEM((1, _TBLK), jnp.float32)],
    )(h, h, sq[:, None, :], sq[:, :, None])


def _edge_kernel(hj_ref, hi_ref, wt_ref, wp_ref, bt_ref, bp_ref,
                 mmax_ref, s_ref):
    blk = hi_ref.shape[1]
    din = hi_ref.shape[2]
    dout = wt_ref.shape[1]
    hi = hi_ref[0]
    hj = hj_ref[0]
    diff = hi[None, :, :] - hj
    t1 = jax.lax.dot_general(diff.reshape(_K * blk, din), wt_ref[...],
                             (((1,), (0,)), ((), ())),
                             preferred_element_type=jnp.float32)
    t2 = jax.lax.dot_general(hi, wp_ref[...], (((1,), (0,)), ((), ())),
                             preferred_element_type=jnp.float32)
    m = ((t1.reshape(_K, blk, dout) + bt_ref[...]) + t2[None, :, :]) \
        + bp_ref[...]
    mmax_ref[0] = jnp.max(m, axis=0)

    step = pl.program_id(0) * pl.num_programs(1) + pl.program_id(1)
    @pl.when(step == 0)
    def _():
        s_ref[...] = jnp.zeros_like(s_ref)
    s1 = jnp.sum(m, axis=(0, 1))
    s2 = jnp.sum(m * m, axis=(0, 1))
    s_ref[...] += jnp.stack([s1, s2], axis=0)


def _edge_mlp(hj, h, wt, wp, bt, bp, blk=256):
    bsub = h.shape[0]
    din = h.shape[-1]
    dout = wt.shape[0]
    grid = (bsub, _N // blk)
    mmax, s = pl.pallas_call(
        _edge_kernel,
        grid=grid,
        in_specs=[
            pl.BlockSpec((1, _K, blk, din), lambda b, n: (b, 0, n, 0)),
            pl.BlockSpec((1, blk, din), lambda b, n: (b, n, 0)),
            pl.BlockSpec((din, dout), lambda b, n: (0, 0)),
            pl.BlockSpec((din, dout), lambda b, n: (0, 0)),
            pl.BlockSpec((1, dout), lambda b, n: (0, 0)),
            pl.BlockSpec((1, dout), lambda b, n: (0, 0)),
        ],
        out_specs=[
            pl.BlockSpec((1, blk, dout), lambda b, n: (b, n, 0)),
            pl.BlockSpec((2, dout), lambda b, n: (0, 0)),
        ],
        out_shape=[
            jax.ShapeDtypeStruct((bsub, _N, dout), jnp.float32),
            jax.ShapeDtypeStruct((2, dout), jnp.float32),
        ],
    )(hj, h, wt.T, wp.T, bt[None, :], bp[None, :])
    return mmax, s


def _bn_kernel(mmax_ref, sa_ref, sb_ref, out_ref):
    dout = mmax_ref.shape[2]
    s = sa_ref[...] + sb_ref[...]
    mu = s[0, :] / _BNK
    var = s[1, :] / _BNK - mu * mu
    hn = (mmax_ref[0] - mu[None, :]) / jnp.sqrt(var + 1e-5)[None, :]
    hn = jnp.where(hn >= 0, hn, 0.2 * hn)
    dpad = out_ref.shape[2]
    if dpad > dout:
        hn = jnp.pad(hn, ((0, 0), (0, dpad - dout)))
    out_ref[0] = hn


def _bn_leaky(mmax, sa, sb, dpad):
    bsub = mmax.shape[0]
    dout = mmax.shape[-1]
    return pl.pallas_call(
        _bn_kernel,
        grid=(bsub,),
        in_specs=[
            pl.BlockSpec((1, _N, dout), lambda b: (b, 0, 0)),
            pl.BlockSpec((2, dout), lambda b: (0, 0)),
            pl.BlockSpec((2, dout), lambda b: (0, 0)),
        ],
        out_specs=pl.BlockSpec((1, _N, dpad), lambda b: (b, 0, 0)),
        out_shape=jax.ShapeDtypeStruct((bsub, _N, dpad), jnp.float32),
    )(mmax, sa, sb)


def _proj_pool_kernel(h_ref, w_ref, b_ref, out_ref):
    h = h_ref[0]
    p = jnp.dot(h, w_ref[...], preferred_element_type=jnp.float32) + b_ref[...]
    pmax = jnp.max(p, axis=0, keepdims=True)
    pavg = jnp.mean(p, axis=0, keepdims=True)
    out_ref[0] = jnp.concatenate([pmax, pavg], axis=1)


def _proj_pool(h, pw, pb):
    out = pl.pallas_call(
        _proj_pool_kernel,
        grid=(_B,),
        in_specs=[
            pl.BlockSpec((1, _N, 512), lambda b: (b, 0, 0)),
            pl.BlockSpec((512, 512), lambda b: (0, 0)),
            pl.BlockSpec((1, 512), lambda b: (0, 0)),
        ],
        out_specs=pl.BlockSpec((1, 1, 1024), lambda b: (b, 0, 0)),
        out_shape=jax.ShapeDtypeStruct((_B, 1, 1024), jnp.float32),
    )(h, pw.T, pb[None, :])
    return out.reshape(_B, 1024)


def kernel(x, params):
    h = jnp.pad(x, ((0, 0), (0, 0), (0, 125)))
    hs = []
    for i, dout in enumerate(_FEATURE_DIMS):
        wt = params['theta_w_%d' % i]
        bt = params['theta_b_%d' % i]
        wp = params['phi_w_%d' % i]
        bp = params['phi_b_%d' % i]
        din = h.shape[-1]
        if wt.shape[1] != din:
            pad = din - wt.shape[1]
            wt = jnp.pad(wt, ((0, 0), (0, pad)))
            wp = jnp.pad(wp, ((0, 0), (0, pad)))

        hflat = h.reshape(_B * _N, din)
        b2 = _B // 2
        hjs, hsubs = [], []
        for half in range(2):
            hsub = h[half * b2:(half + 1) * b2]
            gidx = _topk(hsub, boff=half * b2)
            hj = _sc_gather(hflat, gidx.reshape(_NW, -1, _CH), din)
            hjs.append(hj.reshape(b2, _K, _N, din))
            hsubs.append(hsub)
        mmax0, s0 = _edge_mlp(hjs[0], hsubs[0], wt, wp, bt, bp)
        mmax1, s1 = _edge_mlp(hjs[1], hsubs[1], wt, wp, bt, bp)
        dpad = max(dout, 128)
        h = jnp.concatenate([_bn_leaky(mmax0, s0, s1, dpad),
                             _bn_leaky(mmax1, s0, s1, dpad)], axis=0)
        hs.append(h[:, :, :dout])

    hcat = jnp.concatenate(hs, axis=2)
    return _proj_pool(hcat, params['proj_w'], params['proj_b'])


# speedup vs baseline: 1.6489x; 1.1174x over previous
import functools

import jax
from jax import lax
import jax.numpy as jnp
from jax.experimental import pallas as pl
from jax.experimental.pallas import tpu as pltpu
from jax.experimental.pallas import tpu_sc as plsc

_FEATURE_DIMS = [64, 64, 128, 256]
_K = 16
_B, _N = 8, 1024
_BNK = _B * _N * _K

_NW = 32
_CH = 128


def _sc_gather(hflat, gidx, dpad):
    nch = gidx.shape[1]
    per_w = nch * _CH
    e_total = _NW * per_w
    mesh = plsc.VectorSubcoreMesh(core_axis_name="c", subcore_axis_name="s")

    @functools.partial(
        pl.kernel, mesh=mesh,
        out_type=jax.ShapeDtypeStruct((e_total, dpad), jnp.float32),
        scratch_types=[
            pltpu.VMEM((nch, _CH), jnp.int32),
            pltpu.VMEM((_CH, dpad), jnp.float32),
            pltpu.VMEM((_CH, dpad), jnp.float32),
            pltpu.SemaphoreType.DMA,
            pltpu.SemaphoreType.DMA,
        ],
    )
    def k(h_hbm, idx_hbm, out_hbm, idx_v, buf0, buf1, gsem, wsem):
        wid = lax.axis_index("s") * 2 + lax.axis_index("c")
        base = wid * per_w
        pltpu.sync_copy(idx_hbm.at[wid], idx_v)

        def wait_gather(buf):
            pltpu.make_async_copy(h_hbm.at[pl.ds(0, _CH)], buf, gsem).wait()

        pltpu.async_copy(h_hbm.at[idx_v.at[0]], buf0, gsem)

        def body(jj, _):
            j0 = jj * 2
            pltpu.async_copy(h_hbm.at[idx_v.at[j0 + 1]], buf1, gsem)
            wait_gather(buf0)
            pltpu.async_copy(
                buf0, out_hbm.at[pl.ds(base + j0 * _CH, _CH)], wsem).wait()

            @pl.when(jj + 1 < nch // 2)
            def _():
                pltpu.async_copy(h_hbm.at[idx_v.at[j0 + 2]], buf0, gsem)
            wait_gather(buf1)
            pltpu.async_copy(
                buf1, out_hbm.at[pl.ds(base + (j0 + 1) * _CH, _CH)],
                wsem).wait()
            return 0

        lax.fori_loop(0, nch // 2, body, 0)

    return k(hflat, gidx)


_TBLK = 512
_TCHK = 512


def _topk_kernel(boff, h_ref, hc_ref, sqc_ref, sqcol_ref, idx_ref, d_ref,
                 mn_ref):
    b = pl.program_id(0) + boff
    h = h_ref[0]
    hc = hc_ref[0]
    hh = jax.lax.dot_general(h, hc, (((1,), (1,)), ((), ())),
                             preferred_element_type=jnp.float32)
    sqc_row = sqc_ref[0]
    sq_col = sqcol_ref[0]
    d_ref[...] = (sqc_row - 2.0 * hh) + sq_col
    base = b * _N
    nch = _N // _TCHK
    riota = jax.lax.broadcasted_iota(jnp.int32, (_TCHK, _TBLK), 0)

    mn = jnp.full((1, _TBLK), jnp.inf, jnp.float32)
    for g in range(nch):
        ch = d_ref[pl.ds(g * _TCHK, _TCHK), :]
        mn = jnp.minimum(mn, jnp.min(ch, axis=0, keepdims=True))

    for it in range(_K):
        am = jnp.full((1, _TBLK), _N, jnp.int32)
        for g in range(nch):
            ch = d_ref[pl.ds(g * _TCHK, _TCHK), :]
            cand = jnp.where(ch == mn, riota + g * _TCHK, _N)
            am = jnp.minimum(am, jnp.min(cand, axis=0, keepdims=True))
        idx_ref[0, it, :] = am[0] + base
        if it + 1 < _K:
            mn = jnp.full((1, _TBLK), jnp.inf, jnp.float32)
            for g in range(nch):
                ch = d_ref[pl.ds(g * _TCHK, _TCHK), :]
                masked = jnp.where(riota + g * _TCHK == am, jnp.inf, ch)
                d_ref[pl.ds(g * _TCHK, _TCHK), :] = masked
                mn = jnp.minimum(mn, jnp.min(masked, axis=0, keepdims=True))


def _topk(h, boff=0):
    bsub = h.shape[0]
    din = h.shape[-1]
    sq = jnp.sum(h * h, axis=-1)
    return pl.pallas_call(
        functools.partial(_topk_kernel, boff),
        grid=(bsub, _N // _TBLK),
        in_specs=[
            pl.BlockSpec((1, _N, din), lambda b, c: (b, 0, 0)),
            pl.BlockSpec((1, _TBLK, din), lambda b, c: (b, c, 0)),
            pl.BlockSpec((1, 1, _TBLK), lambda b, c: (b, 0, c)),
            pl.BlockSpec((1, _N, 1), lambda b, c: (b, 0, 0)),
        ],
        out_specs=pl.BlockSpec((1, _K, _TBLK), lambda b, c: (b, 0, c)),
        out_shape=jax.ShapeDtypeStruct((bsub, _K, _N), jnp.int32),
        scratch_shapes=[pltpu.VMEM((_N, _TBLK), jnp.float32),
                        pltpu.VMEM((1, _TBLK), jnp.float32)],
    )(h, h, sq[:, None, :], sq[:, :, None])


def _edge_kernel(hj_ref, hi_ref, wt_ref, wp_ref, bt_ref, bp_ref,
                 mmax_ref, s_ref):
    blk = hi_ref.shape[1]
    din = hi_ref.shape[2]
    dout = wt_ref.shape[1]
    hi = hi_ref[0]
    hj = hj_ref[0]
    diff = hi[None, :, :] - hj
    t1 = jax.lax.dot_general(diff.reshape(_K * blk, din), wt_ref[...],
                             (((1,), (0,)), ((), ())),
                             preferred_element_type=jnp.float32)
    t2 = jax.lax.dot_general(hi, wp_ref[...], (((1,), (0,)), ((), ())),
                             preferred_element_type=jnp.float32)
    m = ((t1.reshape(_K, blk, dout) + bt_ref[...]) + t2[None, :, :]) \
        + bp_ref[...]
    mmax_ref[0] = jnp.max(m, axis=0)

    step = pl.program_id(0) * pl.num_programs(1) + pl.program_id(1)
    @pl.when(step == 0)
    def _():
        s_ref[...] = jnp.zeros_like(s_ref)
    s1 = jnp.sum(m, axis=(0, 1))
    s2 = jnp.sum(m * m, axis=(0, 1))
    s_ref[...] += jnp.stack([s1, s2], axis=0)


def _edge_mlp(hj, h, wt, wp, bt, bp, blk=256):
    bsub = h.shape[0]
    din = h.shape[-1]
    dout = wt.shape[0]
    grid = (bsub, _N // blk)
    mmax, s = pl.pallas_call(
        _edge_kernel,
        grid=grid,
        in_specs=[
            pl.BlockSpec((1, _K, blk, din), lambda b, n: (b, 0, n, 0)),
            pl.BlockSpec((1, blk, din), lambda b, n: (b, n, 0)),
            pl.BlockSpec((din, dout), lambda b, n: (0, 0)),
            pl.BlockSpec((din, dout), lambda b, n: (0, 0)),
            pl.BlockSpec((1, dout), lambda b, n: (0, 0)),
            pl.BlockSpec((1, dout), lambda b, n: (0, 0)),
        ],
        out_specs=[
            pl.BlockSpec((1, blk, dout), lambda b, n: (b, n, 0)),
            pl.BlockSpec((2, dout), lambda b, n: (0, 0)),
        ],
        out_shape=[
            jax.ShapeDtypeStruct((bsub, _N, dout), jnp.float32),
            jax.ShapeDtypeStruct((2, dout), jnp.float32),
        ],
    )(hj, h, wt.T, wp.T, bt[None, :], bp[None, :])
    return mmax, s


def _bn_kernel(mmax_ref, sa_ref, sb_ref, out_ref):
    dout = mmax_ref.shape[2]
    s = sa_ref[...] + sb_ref[...]
    mu = s[0, :] / _BNK
    var = s[1, :] / _BNK - mu * mu
    hn = (mmax_ref[0] - mu[None, :]) / jnp.sqrt(var + 1e-5)[None, :]
    hn = jnp.where(hn >= 0, hn, 0.2 * hn)
    dpad = out_ref.shape[2]
    if dpad > dout:
        hn = jnp.pad(hn, ((0, 0), (0, dpad - dout)))
    out_ref[0] = hn


def _bn_leaky(mmax, sa, sb, dpad):
    bsub = mmax.shape[0]
    dout = mmax.shape[-1]
    return pl.pallas_call(
        _bn_kernel,
        grid=(bsub,),
        in_specs=[
            pl.BlockSpec((1, _N, dout), lambda b: (b, 0, 0)),
            pl.BlockSpec((2, dout), lambda b: (0, 0)),
            pl.BlockSpec((2, dout), lambda b: (0, 0)),
        ],
        out_specs=pl.BlockSpec((1, _N, dpad), lambda b: (b, 0, 0)),
        out_shape=jax.ShapeDtypeStruct((bsub, _N, dpad), jnp.float32),
    )(mmax, sa, sb)


def _proj_pool_kernel(h_ref, w_ref, b_ref, out_ref):
    h = h_ref[0]
    p = jnp.dot(h, w_ref[...], preferred_element_type=jnp.float32) + b_ref[...]
    pmax = jnp.max(p, axis=0, keepdims=True)
    pavg = jnp.mean(p, axis=0, keepdims=True)
    out_ref[0] = jnp.concatenate([pmax, pavg], axis=1)


def _proj_pool(h, pw, pb):
    out = pl.pallas_call(
        _proj_pool_kernel,
        grid=(_B,),
        in_specs=[
            pl.BlockSpec((1, _N, 512), lambda b: (b, 0, 0)),
            pl.BlockSpec((512, 512), lambda b: (0, 0)),
            pl.BlockSpec((1, 512), lambda b: (0, 0)),
        ],
        out_specs=pl.BlockSpec((1, 1, 1024), lambda b: (b, 0, 0)),
        out_shape=jax.ShapeDtypeStruct((_B, 1, 1024), jnp.float32),
    )(h, pw.T, pb[None, :])
    return out.reshape(_B, 1024)


def kernel(x, params):
    h = jnp.pad(x, ((0, 0), (0, 0), (0, 125)))
    hs = []
    for i, dout in enumerate(_FEATURE_DIMS):
        wt = params['theta_w_%d' % i]
        bt = params['theta_b_%d' % i]
        wp = params['phi_w_%d' % i]
        bp = params['phi_b_%d' % i]
        din = h.shape[-1]
        if wt.shape[1] != din:
            pad = din - wt.shape[1]
            wt = jnp.pad(wt, ((0, 0), (0, pad)))
            wp = jnp.pad(wp, ((0, 0), (0, pad)))

        hflat = h.reshape(_B * _N, din)
        b2 = _B // 2
        hjs, hsubs = [], []
        for half in range(2):
            hsub = h[half * b2:(half + 1) * b2]
            gidx = _topk(hsub, boff=half * b2)
            hj = _sc_gather(hflat, gidx.reshape(_NW, -1, _CH), din)
            hjs.append(hj.reshape(b2, _K, _N, din))
            hsubs.append(hsub)
        mmax0, s0 = _edge_mlp(hjs[0], hsubs[0], wt, wp, bt, bp)
        mmax1, s1 = _edge_mlp(hjs[1], hsubs[1], wt, wp, bt, bp)
        dpad = max(dout, 128)
        h = jnp.concatenate([_bn_leaky(mmax0, s0, s1, dpad),
                             _bn_leaky(mmax1, s0, s1, dpad)], axis=0)
        hs.append(h[:, :, :dout])

    hcat = jnp.concatenate(hs, axis=2)
    return _proj_pool(hcat, params['proj_w'], params['proj_b'])
